# baseline (speedup 1.0000x reference)
import functools

import jax
import jax.numpy as jnp
from jax import lax
from jax.experimental import pallas as pl
from jax.experimental.pallas import tpu as pltpu
from jax.experimental.pallas import tpu_sc as plsc

EMBED_NUM = 1000
EMBED_DIM = 64
BATCH = 16384
HIST = 200

_NC = 2
_NS = 16
_NW = _NC * _NS
_BPW = BATCH // _NW
_GE = 4
_SPLITS = ((0, 128), (128, 72))
_NGW = _BPW // _GE
_NI = _NGW // 2


def _sc_gather(idx, table):
    mesh = plsc.VectorSubcoreMesh(core_axis_name="c", subcore_axis_name="s")

    @functools.partial(
        pl.kernel,
        mesh=mesh,
        compiler_params=pltpu.CompilerParams(use_tc_tiling_on_sc=False),
        out_type=jax.ShapeDtypeStruct((BATCH, HIST, EMBED_DIM), jnp.float32),
        scratch_types=[
            pltpu.VMEM((2, _GE, HIST), jnp.int32),
            pltpu.VMEM((2, _GE, HIST, EMBED_DIM), jnp.float32),
            pltpu.SemaphoreType.DMA,
            pltpu.SemaphoreType.DMA,
            pltpu.SemaphoreType.DMA,
            pltpu.SemaphoreType.DMA,
            pltpu.SemaphoreType.DMA,
            pltpu.SemaphoreType.DMA,
        ],
    )
    def k(idx_hbm, table_hbm, out_hbm, idx_v, rows_v, sg0, sg1, sw0, sw1, si0, si1):
        wid = lax.axis_index("s") * _NC + lax.axis_index("c")
        bbase = wid * _BPW
        sg = (sg0, sg1)
        sw = (sw0, sw1)
        si = (si0, si1)

        def elem0(g):
            return bbase + g * _GE

        def fire_gathers(g, b):
            for e in range(_GE):
                for off, ln in _SPLITS:
                    pltpu.async_copy(
                        table_hbm.at[idx_v.at[b].at[e].at[pl.ds(off, ln)]],
                        rows_v.at[b].at[e].at[pl.ds(off, ln)],
                        sg[b],
                    )

        def drain_gathers(b):
            pltpu.make_async_copy(out_hbm.at[pl.ds(0, _GE)], rows_v.at[b], sg[b]).wait()

        def fire_write(g, b):
            pltpu.async_copy(rows_v.at[b], out_hbm.at[pl.ds(elem0(g), _GE)], sw[b])

        def drain_write(b):
            pltpu.make_async_copy(out_hbm.at[pl.ds(0, _GE)], rows_v.at[b], sw[b]).wait()

        def fire_idx(g, b):
            pltpu.async_copy(idx_hbm.at[pl.ds(elem0(g), _GE)], idx_v.at[b], si[b])

        def drain_idx(b):
            pltpu.make_async_copy(idx_hbm.at[pl.ds(0, _GE)], idx_v.at[b], si[b]).wait()

        pltpu.sync_copy(idx_hbm.at[pl.ds(elem0(0), _GE)], idx_v.at[0])

        def body(i, carry):
            ga = 2 * i
            gb = 2 * i + 1

            @pl.when(i >= 1)
            def _():
                drain_write(0)
                drain_idx(0)

            fire_gathers(ga, 0)

            @pl.when(i >= 1)
            def _():
                drain_gathers(1)
                fire_write(gb - 2, 1)

            fire_idx(gb, 1)

            @pl.when(i >= 1)
            def _():
                drain_write(1)

            drain_idx(1)
            fire_gathers(gb, 1)
            drain_gathers(0)
            fire_write(ga, 0)

            @pl.when(i < _NI - 1)
            def _():
                fire_idx(ga + 2, 0)

            return carry

        lax.fori_loop(0, _NI, body, 0)

        drain_gathers(1)
        fire_write(_NGW - 1, 1)
        drain_write(0)
        drain_write(1)

    return k(idx, table)


def kernel(visit_order, pos_embed_weight):
    return _sc_gather(visit_order.astype(jnp.int32), pos_embed_weight)
